# baseline (speedup 1.0000x reference)
import functools

import jax
import jax.numpy as jnp
from jax import lax
from jax.experimental import pallas as pl
from jax.experimental.pallas import tpu as pltpu
from jax.experimental.pallas import tpu_sc as plsc

EMBED = 16
FIELD_OFFSET = 1_000_000
CHUNK = 256


def _sc_gather(x_flat, table128):
    info = plsc.get_sparse_core_info()
    nc, ns, lanes = info.num_cores, info.num_subcores, info.num_lanes
    nw = nc * ns
    n_idx = x_flat.shape[0]
    b_per_w = n_idx // nw
    mesh = plsc.VectorSubcoreMesh(core_axis_name="c", subcore_axis_name="s")

    @functools.partial(
        pl.kernel,
        mesh=mesh,
        out_type=jax.ShapeDtypeStruct((n_idx // 8, 128), jnp.float32),
        scratch_types=[
            pltpu.VMEM((b_per_w,), jnp.int32),
            pltpu.VMEM((b_per_w,), jnp.int32),
            pltpu.VMEM((CHUNK, 128), jnp.float32),
            pltpu.VMEM((b_per_w // 8, 128), jnp.float32),
            pltpu.SemaphoreType.DMA,
        ],
        compiler_params=pltpu.CompilerParams(
            use_tc_tiling_on_sc=True, needs_layout_passes=False),
    )
    def gather_k(x_hbm, table_hbm, out_hbm, idx_v, moff_v, wide_v, out_v, sem):
        wid = lax.axis_index("s") * nc + lax.axis_index("c")
        base = wid * b_per_w
        pltpu.sync_copy(x_hbm.at[pl.ds(base, b_per_w)], idx_v)
        iota = lax.iota(jnp.int32, lanes)
        pat = jnp.where(iota % 2 == 1, FIELD_OFFSET, 0).astype(jnp.int32)
        for t in range(b_per_w // lanes):
            sl = pl.ds(t * lanes, lanes)
            r = idx_v[sl] + pat
            moff_v[sl] = (r & 7) * EMBED
            idx_v[sl] = lax.shift_right_logical(r, 3)

        for chunk in range(b_per_w // CHUNK):
            cb = chunk * CHUNK
            copies = [
                pltpu.async_copy(
                    table_hbm.at[idx_v.at[pl.ds(cb + j * 128, 128)]],
                    wide_v.at[pl.ds(j * 128, 128), :],
                    sem,
                )
                for j in range(CHUNK // 128)
            ]
            for c in copies:
                c.wait()

            def extract_group(g, _, cb=cb):
                loc = iota + g * lanes
                glob = iota + cb + g * lanes
                moff16 = moff_v[pl.ds(cb + g * lanes, lanes)]
                basep = glob * EMBED
                for c in range(EMBED):
                    v = plsc.load_gather(wide_v, [loc, moff16 + c])
                    p = basep + c
                    plsc.store_scatter(
                        out_v, [lax.shift_right_logical(p, 7), p & 127], v)
                return 0

            lax.fori_loop(0, CHUNK // lanes, extract_group, 0)

        pltpu.sync_copy(out_v, out_hbm.at[pl.ds(wid * (b_per_w // 8), b_per_w // 8)])

    return gather_k(x_flat, table128)


def _tc_mlp(h, W1, b1, W2, b2, W3, b3, W4, b4):
    n_rows = h.shape[0]
    blk = 2048
    grid = (n_rows // blk,)

    def mlp_k(h_ref, w1, c1, w2, c2, w3, c3, w4, c4, o_ref):
        a = h_ref[...]
        a = jnp.maximum(
            jnp.dot(a, w1[...], preferred_element_type=jnp.float32) + c1[...], 0.0)
        a = jnp.maximum(
            jnp.dot(a, w2[...], preferred_element_type=jnp.float32) + c2[...], 0.0)
        a = jnp.maximum(
            jnp.dot(a, w3[...], preferred_element_type=jnp.float32) + c3[...], 0.0)
        a = jnp.maximum(
            jnp.dot(a, w4[...], preferred_element_type=jnp.float32) + c4[...], 0.0)
        o_ref[...] = a

    full = lambda arr: pl.BlockSpec(arr.shape, lambda i: (0, 0))
    return pl.pallas_call(
        mlp_k,
        grid=grid,
        in_specs=[
            pl.BlockSpec((blk, 32), lambda i: (i, 0)),
            full(W1), full(b1), full(W2), full(b2),
            full(W3), full(b3), full(W4), full(b4),
        ],
        out_specs=pl.BlockSpec((blk, 1), lambda i: (i, 0)),
        out_shape=jax.ShapeDtypeStruct((n_rows, 1), jnp.float32),
    )(h, W1, b1, W2, b2, W3, b3, W4, b4)


def kernel(x, table, W1, b1, W2, b2, W3, b3, W4, b4):
    n_rows = x.shape[0]
    x_flat = x.reshape(-1)
    table128 = table.reshape(-1, 128)
    packed = _sc_gather(x_flat, table128)
    h = packed.reshape(n_rows, 2 * EMBED)
    return _tc_mlp(
        h,
        W1, b1.reshape(1, -1),
        W2, b2.reshape(1, -1),
        W3, b3.reshape(1, -1),
        W4, b4.reshape(1, -1),
    )


# speedup vs baseline: 16.1433x; 16.1433x over previous
import functools

import jax
import jax.numpy as jnp
from jax import lax
from jax.experimental import pallas as pl
from jax.experimental.pallas import tpu as pltpu
from jax.experimental.pallas import tpu_sc as plsc

EMBED = 16
FIELD_OFFSET = 1_000_000
HALF_STRIDE = 16_000_000


def _sc_gather_t(x_t, table_flat, n_rows):
    info = plsc.get_sparse_core_info()
    nc, ns, lanes = info.num_cores, info.num_subcores, info.num_lanes
    nw = nc * ns
    mesh = plsc.VectorSubcoreMesh(core_axis_name="c", subcore_axis_name="s")

    @functools.partial(
        pl.kernel,
        mesh=mesh,
        out_type=jax.ShapeDtypeStruct((nw, n_rows), jnp.float32),
        scratch_types=[
            pltpu.VMEM((n_rows,), jnp.int32),
            pltpu.VMEM((n_rows,), jnp.int32),
            pltpu.VMEM((n_rows,), jnp.float32),
            pltpu.SemaphoreType.DMA,
        ],
        compiler_params=pltpu.CompilerParams(
            use_tc_tiling_on_sc=True, needs_layout_passes=False),
    )
    def gather_k(x_hbm, tab_hbm, out_hbm, xrow_v, idx_v, dst_v, sem):
        w = lax.axis_index("s") * nc + lax.axis_index("c")
        field = w // EMBED
        f = w % EMBED
        pltpu.sync_copy(x_hbm.at[field], xrow_v)
        base = (f // 8) * HALF_STRIDE + (f % 8) * 128
        roff = field * FIELD_OFFSET

        def addr_chunk(t, _):
            sl = pl.ds(t * lanes, lanes)
            r = xrow_v[sl] + roff
            idx_v[sl] = (lax.shift_left(lax.shift_right_logical(r, 7), 10)
                         + (r & 127) + base)
            return 0

        lax.fori_loop(0, n_rows // lanes, addr_chunk, 0)

        def fire_chunk(j, _):
            copies = [
                pltpu.async_copy(
                    tab_hbm.at[idx_v.at[pl.ds(j * 2048 + k * 128, 128)]],
                    dst_v.at[pl.ds(j * 2048 + k * 128, 128)],
                    sem,
                )
                for k in range(16)
            ]
            for c in copies:
                c.wait()
            return 0

        lax.fori_loop(0, n_rows // 2048, fire_chunk, 0)
        pltpu.sync_copy(dst_v, out_hbm.at[w])

    return gather_k(x_t, table_flat)


def _tc_mlp_t(h_t, W1t, b1, W2t, b2, W3t, b3, W4t, b4):
    n_rows = h_t.shape[1]
    blk = 2048
    grid = (n_rows // blk,)

    def mlp_k(h_ref, w1, c1, w2, c2, w3, c3, w4, c4, o_ref):
        a = h_ref[...]
        a = jnp.maximum(
            jnp.dot(w1[...], a, preferred_element_type=jnp.float32) + c1[...], 0.0)
        a = jnp.maximum(
            jnp.dot(w2[...], a, preferred_element_type=jnp.float32) + c2[...], 0.0)
        a = jnp.maximum(
            jnp.dot(w3[...], a, preferred_element_type=jnp.float32) + c3[...], 0.0)
        a = jnp.maximum(
            jnp.dot(w4[...], a, preferred_element_type=jnp.float32) + c4[...], 0.0)
        o_ref[...] = a

    full = lambda arr: pl.BlockSpec(arr.shape, lambda i: (0, 0))
    return pl.pallas_call(
        mlp_k,
        grid=grid,
        in_specs=[
            pl.BlockSpec((2 * EMBED, blk), lambda i: (0, i)),
            full(W1t), full(b1), full(W2t), full(b2),
            full(W3t), full(b3), full(W4t), full(b4),
        ],
        out_specs=pl.BlockSpec((1, blk), lambda i: (0, i)),
        out_shape=jax.ShapeDtypeStruct((1, n_rows), jnp.float32),
    )(h_t, W1t, b1, W2t, b2, W3t, b3, W4t, b4)


def kernel(x, table, W1, b1, W2, b2, W3, b3, W4, b4):
    n_rows = x.shape[0]
    x_t = x.T
    table_flat = (table.T.reshape(2, 8, 15625, 128)
                  .transpose(0, 2, 1, 3).reshape(-1))
    h_t = _sc_gather_t(x_t, table_flat, n_rows)
    out_t = _tc_mlp_t(
        h_t,
        W1.T, b1.reshape(-1, 1),
        W2.T, b2.reshape(-1, 1),
        W3.T, b3.reshape(-1, 1),
        W4.T, b4.reshape(-1, 1),
    )
    return out_t.reshape(n_rows, 1)


# speedup vs baseline: 17.1058x; 1.0596x over previous
import functools

import jax
import jax.numpy as jnp
from jax import lax
from jax.experimental import pallas as pl
from jax.experimental.pallas import tpu as pltpu
from jax.experimental.pallas import tpu_sc as plsc

EMBED = 16
FIELD_OFFSET = 1_000_000
HALF_STRIDE = 16_000_000


def _sc_gather_t(x_t, table_flat, n_rows):
    info = plsc.get_sparse_core_info()
    nc, ns, lanes = info.num_cores, info.num_subcores, info.num_lanes
    nw = nc * ns
    mesh = plsc.VectorSubcoreMesh(core_axis_name="c", subcore_axis_name="s")

    @functools.partial(
        pl.kernel,
        mesh=mesh,
        out_type=jax.ShapeDtypeStruct((nw, n_rows), jnp.float32),
        scratch_types=[
            pltpu.VMEM((n_rows,), jnp.int32),
            pltpu.VMEM((n_rows,), jnp.int32),
            pltpu.VMEM((n_rows,), jnp.float32),
            pltpu.SemaphoreType.DMA,
        ],
        compiler_params=pltpu.CompilerParams(
            use_tc_tiling_on_sc=True, needs_layout_passes=False),
    )
    def gather_k(x_hbm, tab_hbm, out_hbm, xrow_v, idx_v, dst_v, sem):
        w = lax.axis_index("s") * nc + lax.axis_index("c")
        field = w // EMBED
        f = w % EMBED
        pltpu.sync_copy(x_hbm.at[field], xrow_v)
        base = (f // 8) * HALF_STRIDE + (f % 8) * 128
        roff = field * FIELD_OFFSET

        def addr_chunk(t, _):
            for u in range(4):
                sl = pl.ds((t * 4 + u) * lanes, lanes)
                r = xrow_v[sl] + roff
                idx_v[sl] = (lax.shift_left(lax.shift_right_logical(r, 7), 10)
                             + (r & 127) + base)
            return 0

        lax.fori_loop(0, n_rows // (4 * lanes), addr_chunk, 0)

        n_chunks = n_rows // 2048

        def fire_chunk(j, _):
            for k in range(16):
                pltpu.async_copy(
                    tab_hbm.at[idx_v.at[pl.ds(j * 2048 + k * 128, 128)]],
                    dst_v.at[pl.ds(j * 2048 + k * 128, 128)],
                    sem,
                )

            @pl.when(j > 0)
            def _():
                for k in range(16):
                    pltpu.make_async_copy(
                        tab_hbm.at[idx_v.at[pl.ds(0, 128)]],
                        dst_v.at[pl.ds((j - 1) * 2048 + k * 128, 128)],
                        sem,
                    ).wait()

            return 0

        lax.fori_loop(0, n_chunks, fire_chunk, 0)
        for k in range(16):
            pltpu.make_async_copy(
                tab_hbm.at[idx_v.at[pl.ds(0, 128)]],
                dst_v.at[pl.ds((n_chunks - 1) * 2048 + k * 128, 128)],
                sem,
            ).wait()
        pltpu.sync_copy(dst_v, out_hbm.at[w])

    return gather_k(x_t, table_flat)


def _tc_mlp_t(h_t, W1t, b1, W2t, b2, W3t, b3, W4t, b4):
    n_rows = h_t.shape[1]
    blk = 2048
    grid = (n_rows // blk,)

    def mlp_k(h_ref, w1, c1, w2, c2, w3, c3, w4, c4, o_ref):
        a = h_ref[...]
        a = jnp.maximum(
            jnp.dot(w1[...], a, preferred_element_type=jnp.float32) + c1[...], 0.0)
        a = jnp.maximum(
            jnp.dot(w2[...], a, preferred_element_type=jnp.float32) + c2[...], 0.0)
        a = jnp.maximum(
            jnp.dot(w3[...], a, preferred_element_type=jnp.float32) + c3[...], 0.0)
        a = jnp.maximum(
            jnp.dot(w4[...], a, preferred_element_type=jnp.float32) + c4[...], 0.0)
        o_ref[...] = a

    full = lambda arr: pl.BlockSpec(arr.shape, lambda i: (0, 0))
    return pl.pallas_call(
        mlp_k,
        grid=grid,
        in_specs=[
            pl.BlockSpec((2 * EMBED, blk), lambda i: (0, i)),
            full(W1t), full(b1), full(W2t), full(b2),
            full(W3t), full(b3), full(W4t), full(b4),
        ],
        out_specs=pl.BlockSpec((1, blk), lambda i: (0, i)),
        out_shape=jax.ShapeDtypeStruct((1, n_rows), jnp.float32),
    )(h_t, W1t, b1, W2t, b2, W3t, b3, W4t, b4)


def kernel(x, table, W1, b1, W2, b2, W3, b3, W4, b4):
    n_rows = x.shape[0]
    x_t = x.T
    table_flat = (table.T.reshape(2, 8, 15625, 128)
                  .transpose(0, 2, 1, 3).reshape(-1))
    h_t = _sc_gather_t(x_t, table_flat, n_rows)
    out_t = _tc_mlp_t(
        h_t,
        W1.T, b1.reshape(-1, 1),
        W2.T, b2.reshape(-1, 1),
        W3.T, b3.reshape(-1, 1),
        W4.T, b4.reshape(-1, 1),
    )
    return out_t.reshape(n_rows, 1)


# speedup vs baseline: 17.4750x; 1.0216x over previous
import functools

import jax
import jax.numpy as jnp
from jax import lax
from jax.experimental import pallas as pl
from jax.experimental.pallas import tpu as pltpu
from jax.experimental.pallas import tpu_sc as plsc

EMBED = 16
FIELD_OFFSET = 1_000_000
HALF_STRIDE = 16_000_000


def _sc_gather_t(x_t, table_flat, n_rows):
    info = plsc.get_sparse_core_info()
    nc, ns, lanes = info.num_cores, info.num_subcores, info.num_lanes
    nw = nc * ns
    mesh = plsc.VectorSubcoreMesh(core_axis_name="c", subcore_axis_name="s")

    @functools.partial(
        pl.kernel,
        mesh=mesh,
        out_type=jax.ShapeDtypeStruct((nw, n_rows), jnp.float32),
        scratch_types=[
            pltpu.VMEM((n_rows,), jnp.int32),
            pltpu.VMEM((n_rows,), jnp.int32),
            pltpu.VMEM((n_rows,), jnp.float32),
            pltpu.SemaphoreType.DMA,
        ],
        compiler_params=pltpu.CompilerParams(
            use_tc_tiling_on_sc=True, needs_layout_passes=False),
    )
    def gather_k(x_hbm, tab_hbm, out_hbm, xrow_v, idx_v, dst_v, sem):
        w = lax.axis_index("s") * nc + lax.axis_index("c")
        field = w // EMBED
        f = w % EMBED
        pltpu.sync_copy(x_hbm.at[field], xrow_v)
        base = (f // 8) * HALF_STRIDE + (f % 8) * 128
        roff = field * FIELD_OFFSET

        n_chunks = n_rows // 2048
        groups_per_chunk = 2048 // lanes

        def addr_block(j):
            def inner(t, _):
                for u in range(4):
                    sl = pl.ds(j * 2048 + (t * 4 + u) * lanes, lanes)
                    r = xrow_v[sl] + roff
                    idx_v[sl] = (
                        lax.shift_left(lax.shift_right_logical(r, 7), 10)
                        + (r & 127) + base)
                return 0

            lax.fori_loop(0, groups_per_chunk // 4, inner, 0)

        addr_block(0)

        def fire_chunk(j, _):
            for k in range(16):
                pltpu.async_copy(
                    tab_hbm.at[idx_v.at[pl.ds(j * 2048 + k * 128, 128)]],
                    dst_v.at[pl.ds(j * 2048 + k * 128, 128)],
                    sem,
                )

            @pl.when(j < n_chunks - 1)
            def _():
                addr_block(j + 1)

            @pl.when(j > 0)
            def _():
                for k in range(16):
                    pltpu.make_async_copy(
                        tab_hbm.at[idx_v.at[pl.ds(0, 128)]],
                        dst_v.at[pl.ds((j - 1) * 2048 + k * 128, 128)],
                        sem,
                    ).wait()

            return 0

        lax.fori_loop(0, n_chunks, fire_chunk, 0)
        for k in range(16):
            pltpu.make_async_copy(
                tab_hbm.at[idx_v.at[pl.ds(0, 128)]],
                dst_v.at[pl.ds((n_chunks - 1) * 2048 + k * 128, 128)],
                sem,
            ).wait()
        pltpu.sync_copy(dst_v, out_hbm.at[w])

    return gather_k(x_t, table_flat)


def _tc_mlp_t(h_t, W1t, b1, W2t, b2, W3t, b3, W4t, b4):
    n_rows = h_t.shape[1]
    blk = 2048
    grid = (n_rows // blk,)

    def mlp_k(h_ref, w1, c1, w2, c2, w3, c3, w4, c4, o_ref):
        a = h_ref[...]
        a = jnp.maximum(
            jnp.dot(w1[...], a, preferred_element_type=jnp.float32) + c1[...], 0.0)
        a = jnp.maximum(
            jnp.dot(w2[...], a, preferred_element_type=jnp.float32) + c2[...], 0.0)
        a = jnp.maximum(
            jnp.dot(w3[...], a, preferred_element_type=jnp.float32) + c3[...], 0.0)
        a = jnp.maximum(
            jnp.dot(w4[...], a, preferred_element_type=jnp.float32) + c4[...], 0.0)
        o_ref[...] = a

    full = lambda arr: pl.BlockSpec(arr.shape, lambda i: (0, 0))
    return pl.pallas_call(
        mlp_k,
        grid=grid,
        in_specs=[
            pl.BlockSpec((2 * EMBED, blk), lambda i: (0, i)),
            full(W1t), full(b1), full(W2t), full(b2),
            full(W3t), full(b3), full(W4t), full(b4),
        ],
        out_specs=pl.BlockSpec((1, blk), lambda i: (0, i)),
        out_shape=jax.ShapeDtypeStruct((1, n_rows), jnp.float32),
    )(h_t, W1t, b1, W2t, b2, W3t, b3, W4t, b4)


def kernel(x, table, W1, b1, W2, b2, W3, b3, W4, b4):
    n_rows = x.shape[0]
    x_t = x.T
    table_flat = (table.T.reshape(2, 8, 15625, 128)
                  .transpose(0, 2, 1, 3).reshape(-1))
    h_t = _sc_gather_t(x_t, table_flat, n_rows)
    out_t = _tc_mlp_t(
        h_t,
        W1.T, b1.reshape(-1, 1),
        W2.T, b2.reshape(-1, 1),
        W3.T, b3.reshape(-1, 1),
        W4.T, b4.reshape(-1, 1),
    )
    return out_t.reshape(n_rows, 1)


# speedup vs baseline: 18.6648x; 1.0681x over previous
import functools

import jax
import jax.numpy as jnp
from jax import lax
from jax.experimental import pallas as pl
from jax.experimental.pallas import tpu as pltpu
from jax.experimental.pallas import tpu_sc as plsc

EMBED = 16
FIELD_OFFSET = 1_000_000
HALF_STRIDE = 16_000_000


def _sc_gather_t(x_t, table_flat, n_rows):
    info = plsc.get_sparse_core_info()
    nc, ns, lanes = info.num_cores, info.num_subcores, info.num_lanes
    nw = nc * ns
    mesh = plsc.VectorSubcoreMesh(core_axis_name="c", subcore_axis_name="s")

    @functools.partial(
        pl.kernel,
        mesh=mesh,
        out_type=jax.ShapeDtypeStruct((nw, n_rows), jnp.float32),
        scratch_types=[
            pltpu.VMEM((n_rows,), jnp.int32),
            pltpu.VMEM((n_rows,), jnp.int32),
            pltpu.VMEM((n_rows,), jnp.float32),
            pltpu.SemaphoreType.DMA,
        ],
        compiler_params=pltpu.CompilerParams(
            use_tc_tiling_on_sc=True, needs_layout_passes=False),
    )
    def gather_k(x_hbm, tab_hbm, out_hbm, xrow_v, idx_v, dst_v, sem):
        w = lax.axis_index("s") * nc + lax.axis_index("c")
        field = w // EMBED
        f = w % EMBED
        pltpu.sync_copy(x_hbm.at[field], xrow_v)
        base = (f // 8) * HALF_STRIDE + (f % 8) * 128
        roff = field * FIELD_OFFSET

        n_chunks = n_rows // 2048
        groups_per_chunk = 2048 // lanes

        def addr_block(j):
            def inner(t, _):
                for u in range(4):
                    sl = pl.ds(j * 2048 + (t * 4 + u) * lanes, lanes)
                    r = xrow_v[sl] + roff
                    idx_v[sl] = (
                        lax.shift_left(lax.shift_right_logical(r, 7), 10)
                        + (r & 127) + base)
                return 0

            lax.fori_loop(0, groups_per_chunk // 4, inner, 0)

        addr_block(0)

        def fire_chunk(j, _):
            for k in range(16):
                pltpu.async_copy(
                    tab_hbm.at[idx_v.at[pl.ds(j * 2048 + k * 128, 128)]],
                    dst_v.at[pl.ds(j * 2048 + k * 128, 128)],
                    sem,
                )

            @pl.when(j < n_chunks - 1)
            def _():
                addr_block(j + 1)

            @pl.when(j > 1)
            def _():
                for k in range(16):
                    pltpu.make_async_copy(
                        tab_hbm.at[idx_v.at[pl.ds(0, 128)]],
                        dst_v.at[pl.ds((j - 2) * 2048 + k * 128, 128)],
                        sem,
                    ).wait()

            return 0

        lax.fori_loop(0, n_chunks, fire_chunk, 0)
        for j in (n_chunks - 2, n_chunks - 1):
            for k in range(16):
                pltpu.make_async_copy(
                    tab_hbm.at[idx_v.at[pl.ds(0, 128)]],
                    dst_v.at[pl.ds(j * 2048 + k * 128, 128)],
                    sem,
                ).wait()
        pltpu.sync_copy(dst_v, out_hbm.at[w])

    return gather_k(x_t, table_flat)


def _tc_mlp_t(h_t, W1t, b1, W2t, b2, W3t, b3, W4t, b4):
    n_rows = h_t.shape[1]
    blk = 4096
    grid = (n_rows // blk,)

    def mlp_k(h_ref, w1, c1, w2, c2, w3, c3, w4, c4, o_ref):
        a = h_ref[...]
        a = jnp.maximum(
            jnp.dot(w1[...], a, preferred_element_type=jnp.float32) + c1[...], 0.0)
        a = jnp.maximum(
            jnp.dot(w2[...], a, preferred_element_type=jnp.float32) + c2[...], 0.0)
        a = jnp.maximum(
            jnp.dot(w3[...], a, preferred_element_type=jnp.float32) + c3[...], 0.0)
        a = jnp.maximum(
            jnp.dot(w4[...], a, preferred_element_type=jnp.float32) + c4[...], 0.0)
        o_ref[...] = a

    full = lambda arr: pl.BlockSpec(arr.shape, lambda i: (0, 0))
    return pl.pallas_call(
        mlp_k,
        grid=grid,
        in_specs=[
            pl.BlockSpec((2 * EMBED, blk), lambda i: (0, i)),
            full(W1t), full(b1), full(W2t), full(b2),
            full(W3t), full(b3), full(W4t), full(b4),
        ],
        out_specs=pl.BlockSpec((1, blk), lambda i: (0, i)),
        out_shape=jax.ShapeDtypeStruct((1, n_rows), jnp.float32),
    )(h_t, W1t, b1, W2t, b2, W3t, b3, W4t, b4)


def kernel(x, table, W1, b1, W2, b2, W3, b3, W4, b4):
    n_rows = x.shape[0]
    x_t = x.T
    table_flat = (table.T.reshape(2, 8, 15625, 128)
                  .transpose(0, 2, 1, 3).reshape(-1))
    h_t = _sc_gather_t(x_t, table_flat, n_rows)
    out_t = _tc_mlp_t(
        h_t,
        W1.T, b1.reshape(-1, 1),
        W2.T, b2.reshape(-1, 1),
        W3.T, b3.reshape(-1, 1),
        W4.T, b4.reshape(-1, 1),
    )
    return out_t.reshape(n_rows, 1)


# speedup vs baseline: 19.2335x; 1.0305x over previous
import functools

import jax
import jax.numpy as jnp
from jax import lax
from jax.experimental import pallas as pl
from jax.experimental.pallas import tpu as pltpu
from jax.experimental.pallas import tpu_sc as plsc

EMBED = 16
FIELD_OFFSET = 1_000_000
HALF_STRIDE = 16_000_000


def _sc_gather_t(x_t, table_flat, n_rows):
    info = plsc.get_sparse_core_info()
    nc, ns, lanes = info.num_cores, info.num_subcores, info.num_lanes
    nw = nc * ns
    mesh = plsc.VectorSubcoreMesh(core_axis_name="c", subcore_axis_name="s")

    @functools.partial(
        pl.kernel,
        mesh=mesh,
        out_type=jax.ShapeDtypeStruct((nw, n_rows), jnp.float32),
        scratch_types=[
            pltpu.VMEM((n_rows,), jnp.int32),
            pltpu.VMEM((n_rows,), jnp.int32),
            pltpu.VMEM((n_rows,), jnp.float32),
            pltpu.SemaphoreType.DMA,
        ],
        compiler_params=pltpu.CompilerParams(
            use_tc_tiling_on_sc=True, needs_layout_passes=False),
    )
    def gather_k(x_hbm, tab_hbm, out_hbm, xrow_v, idx_v, dst_v, sem):
        w = lax.axis_index("s") * nc + lax.axis_index("c")
        field = w // EMBED
        f = w % EMBED
        pltpu.sync_copy(x_hbm.at[field], xrow_v)
        base = (f // 8) * HALF_STRIDE + (f % 8) * 128
        roff = field * FIELD_OFFSET

        n_chunks = n_rows // 2048
        groups_per_chunk = 2048 // lanes

        def addr_block(j):
            def inner(t, _):
                for u in range(4):
                    sl = pl.ds(j * 2048 + (t * 4 + u) * lanes, lanes)
                    r = xrow_v[sl] + roff
                    idx_v[sl] = (
                        lax.shift_left(lax.shift_right_logical(r, 7), 10)
                        + (r & 127) + base)
                return 0

            lax.fori_loop(0, groups_per_chunk // 4, inner, 0)

        addr_block(0)

        def fire_chunk(j, _):
            for k in range(16):
                pltpu.async_copy(
                    tab_hbm.at[idx_v.at[pl.ds(j * 2048 + k * 128, 128)]],
                    dst_v.at[pl.ds(j * 2048 + k * 128, 128)],
                    sem,
                )

            @pl.when(j < n_chunks - 1)
            def _():
                addr_block(j + 1)

            @pl.when(j > 3)
            def _():
                for k in range(16):
                    pltpu.make_async_copy(
                        tab_hbm.at[idx_v.at[pl.ds(0, 128)]],
                        dst_v.at[pl.ds((j - 4) * 2048 + k * 128, 128)],
                        sem,
                    ).wait()

            return 0

        lax.fori_loop(0, n_chunks, fire_chunk, 0)
        for j in range(n_chunks - 4, n_chunks):
            for k in range(16):
                pltpu.make_async_copy(
                    tab_hbm.at[idx_v.at[pl.ds(0, 128)]],
                    dst_v.at[pl.ds(j * 2048 + k * 128, 128)],
                    sem,
                ).wait()
        pltpu.sync_copy(dst_v, out_hbm.at[w])

    return gather_k(x_t, table_flat)


def _tc_mlp_t(h_t, W1t, b1, W2t, b2, W3t, b3, W4t, b4):
    n_rows = h_t.shape[1]
    blk = 4096
    grid = (n_rows // blk,)

    def mlp_k(h_ref, w1, c1, w2, c2, w3, c3, w4, c4, o_ref):
        a = h_ref[...]
        a = jnp.maximum(
            jnp.dot(w1[...], a, preferred_element_type=jnp.float32) + c1[...], 0.0)
        a = jnp.maximum(
            jnp.dot(w2[...], a, preferred_element_type=jnp.float32) + c2[...], 0.0)
        a = jnp.maximum(
            jnp.dot(w3[...], a, preferred_element_type=jnp.float32) + c3[...], 0.0)
        a = jnp.maximum(
            jnp.dot(w4[...], a, preferred_element_type=jnp.float32) + c4[...], 0.0)
        o_ref[...] = a

    full = lambda arr: pl.BlockSpec(arr.shape, lambda i: (0, 0))
    return pl.pallas_call(
        mlp_k,
        grid=grid,
        in_specs=[
            pl.BlockSpec((2 * EMBED, blk), lambda i: (0, i)),
            full(W1t), full(b1), full(W2t), full(b2),
            full(W3t), full(b3), full(W4t), full(b4),
        ],
        out_specs=pl.BlockSpec((1, blk), lambda i: (0, i)),
        out_shape=jax.ShapeDtypeStruct((1, n_rows), jnp.float32),
    )(h_t, W1t, b1, W2t, b2, W3t, b3, W4t, b4)


def kernel(x, table, W1, b1, W2, b2, W3, b3, W4, b4):
    n_rows = x.shape[0]
    x_t = x.T
    table_flat = (table.T.reshape(2, 8, 15625, 128)
                  .transpose(0, 2, 1, 3).reshape(-1))
    h_t = _sc_gather_t(x_t, table_flat, n_rows)
    out_t = _tc_mlp_t(
        h_t,
        W1.T, b1.reshape(-1, 1),
        W2.T, b2.reshape(-1, 1),
        W3.T, b3.reshape(-1, 1),
        W4.T, b4.reshape(-1, 1),
    )
    return out_t.reshape(n_rows, 1)


# speedup vs baseline: 19.8722x; 1.0332x over previous
import functools

import jax
import jax.numpy as jnp
from jax import lax
from jax.experimental import pallas as pl
from jax.experimental.pallas import tpu as pltpu
from jax.experimental.pallas import tpu_sc as plsc

EMBED = 16
FIELD_OFFSET = 1_000_000
HALF_STRIDE = 16_000_000


def _sc_gather_t(x_t, table_flat, n_rows):
    info = plsc.get_sparse_core_info()
    nc, ns, lanes = info.num_cores, info.num_subcores, info.num_lanes
    nw = nc * ns
    mesh = plsc.VectorSubcoreMesh(core_axis_name="c", subcore_axis_name="s")

    @functools.partial(
        pl.kernel,
        mesh=mesh,
        out_type=jax.ShapeDtypeStruct((nw, n_rows), jnp.float32),
        scratch_types=[
            pltpu.VMEM((n_rows,), jnp.int32),
            pltpu.VMEM((n_rows,), jnp.int32),
            pltpu.VMEM((n_rows,), jnp.float32),
            pltpu.SemaphoreType.DMA,
        ],
        compiler_params=pltpu.CompilerParams(
            use_tc_tiling_on_sc=True, needs_layout_passes=False),
    )
    def gather_k(x_hbm, tab_hbm, out_hbm, xrow_v, idx_v, dst_v, sem):
        w = lax.axis_index("s") * nc + lax.axis_index("c")
        field = w // EMBED
        f = w % EMBED
        pltpu.sync_copy(x_hbm.at[field], xrow_v)
        base = (f // 8) * HALF_STRIDE + (f % 8) * 128
        roff = field * FIELD_OFFSET

        n_chunks = n_rows // 2048
        groups_per_chunk = 2048 // lanes

        def addr_block(j):
            def inner(t, _):
                for u in range(4):
                    sl = pl.ds(j * 2048 + (t * 4 + u) * lanes, lanes)
                    r = xrow_v[sl] + roff
                    idx_v[sl] = (
                        lax.shift_left(lax.shift_right_logical(r, 7), 10)
                        + (r & 127) + base)
                return 0

            lax.fori_loop(0, groups_per_chunk // 4, inner, 0)

        addr_block(0)

        def fire_chunk(j, _):
            for k in range(16):
                pltpu.async_copy(
                    tab_hbm.at[idx_v.at[pl.ds(j * 2048 + k * 128, 128)]],
                    dst_v.at[pl.ds(j * 2048 + k * 128, 128)],
                    sem,
                )

            @pl.when(j < n_chunks - 1)
            def _():
                addr_block(j + 1)

            return 0

        lax.fori_loop(0, n_chunks, fire_chunk, 0)

        def drain_chunk(j, _):
            for k in range(16):
                pltpu.make_async_copy(
                    tab_hbm.at[idx_v.at[pl.ds(0, 128)]],
                    dst_v.at[pl.ds(j * 2048 + k * 128, 128)],
                    sem,
                ).wait()
            return 0

        lax.fori_loop(0, n_chunks, drain_chunk, 0)
        pltpu.sync_copy(dst_v, out_hbm.at[w])

    return gather_k(x_t, table_flat)


def _tc_mlp_t(h_t, W1t, b1, W2t, b2, W3t, b3, W4t, b4):
    n_rows = h_t.shape[1]
    blk = 4096
    grid = (n_rows // blk,)

    def mlp_k(h_ref, w1, c1, w2, c2, w3, c3, w4, c4, o_ref):
        a = h_ref[...]
        a = jnp.maximum(
            jnp.dot(w1[...], a, preferred_element_type=jnp.float32) + c1[...], 0.0)
        a = jnp.maximum(
            jnp.dot(w2[...], a, preferred_element_type=jnp.float32) + c2[...], 0.0)
        a = jnp.maximum(
            jnp.dot(w3[...], a, preferred_element_type=jnp.float32) + c3[...], 0.0)
        a = jnp.maximum(
            jnp.dot(w4[...], a, preferred_element_type=jnp.float32) + c4[...], 0.0)
        o_ref[...] = a

    full = lambda arr: pl.BlockSpec(arr.shape, lambda i: (0, 0))
    return pl.pallas_call(
        mlp_k,
        grid=grid,
        in_specs=[
            pl.BlockSpec((2 * EMBED, blk), lambda i: (0, i)),
            full(W1t), full(b1), full(W2t), full(b2),
            full(W3t), full(b3), full(W4t), full(b4),
        ],
        out_specs=pl.BlockSpec((1, blk), lambda i: (0, i)),
        out_shape=jax.ShapeDtypeStruct((1, n_rows), jnp.float32),
    )(h_t, W1t, b1, W2t, b2, W3t, b3, W4t, b4)


def kernel(x, table, W1, b1, W2, b2, W3, b3, W4, b4):
    n_rows = x.shape[0]
    x_t = x.T
    table_flat = (table.T.reshape(2, 8, 15625, 128)
                  .transpose(0, 2, 1, 3).reshape(-1))
    h_t = _sc_gather_t(x_t, table_flat, n_rows)
    out_t = _tc_mlp_t(
        h_t,
        W1.T, b1.reshape(-1, 1),
        W2.T, b2.reshape(-1, 1),
        W3.T, b3.reshape(-1, 1),
        W4.T, b4.reshape(-1, 1),
    )
    return out_t.reshape(n_rows, 1)


# speedup vs baseline: 20.4734x; 1.0303x over previous
import functools

import jax
import jax.numpy as jnp
from jax import lax
from jax.experimental import pallas as pl
from jax.experimental.pallas import tpu as pltpu
from jax.experimental.pallas import tpu_sc as plsc

EMBED = 16
FIELD_OFFSET = 1_000_000
HALF_STRIDE = 16_000_000


def _sc_gather_t(x_t, table_flat, n_rows):
    info = plsc.get_sparse_core_info()
    nc, ns, lanes = info.num_cores, info.num_subcores, info.num_lanes
    nw = nc * ns
    mesh = plsc.VectorSubcoreMesh(core_axis_name="c", subcore_axis_name="s")

    @functools.partial(
        pl.kernel,
        mesh=mesh,
        out_type=jax.ShapeDtypeStruct((nw, n_rows), jnp.float32),
        scratch_types=[
            pltpu.VMEM((n_rows,), jnp.int32),
            pltpu.VMEM((n_rows,), jnp.int32),
            pltpu.VMEM((n_rows,), jnp.float32),
            pltpu.SemaphoreType.DMA,
        ],
        compiler_params=pltpu.CompilerParams(
            use_tc_tiling_on_sc=True, needs_layout_passes=False),
    )
    def gather_k(x_hbm, tab_hbm, out_hbm, xrow_v, idx_v, dst_v, sem):
        w = lax.axis_index("s") * nc + lax.axis_index("c")
        field = w // EMBED
        f = w % EMBED
        pltpu.sync_copy(x_hbm.at[field], xrow_v)
        base = (f // 8) * HALF_STRIDE + (f % 8) * 128
        roff = field * FIELD_OFFSET

        chunk = 1024
        n_chunks = n_rows // chunk
        groups_per_chunk = chunk // lanes

        def addr_block(j):
            def inner(t, _):
                for u in range(4):
                    sl = pl.ds(j * chunk + (t * 4 + u) * lanes, lanes)
                    r = xrow_v[sl] + roff
                    idx_v[sl] = (
                        lax.shift_left(lax.shift_right_logical(r, 7), 10)
                        + (r & 127) + base)
                return 0

            lax.fori_loop(0, groups_per_chunk // 4, inner, 0)

        addr_block(0)

        def fire_chunk(j, _):
            for k in range(8):
                pltpu.async_copy(
                    tab_hbm.at[idx_v.at[pl.ds(j * chunk + k * 128, 128)]],
                    dst_v.at[pl.ds(j * chunk + k * 128, 128)],
                    sem,
                )

            @pl.when(j < n_chunks - 1)
            def _():
                addr_block(j + 1)

            return 0

        lax.fori_loop(0, n_chunks, fire_chunk, 0)

        def drain_chunk(j, _):
            for k in range(8):
                pltpu.make_async_copy(
                    tab_hbm.at[idx_v.at[pl.ds(0, 128)]],
                    dst_v.at[pl.ds(j * chunk + k * 128, 128)],
                    sem,
                ).wait()
            return 0

        lax.fori_loop(0, n_chunks, drain_chunk, 0)
        pltpu.sync_copy(dst_v, out_hbm.at[w])

    return gather_k(x_t, table_flat)


def _tc_mlp_t(h_t, W1t, b1, W2t, b2, W3t, b3, W4t, b4):
    n_rows = h_t.shape[1]
    blk = 8192
    grid = (n_rows // blk,)

    def mlp_k(h_ref, w1, c1, w2, c2, w3, c3, w4, c4, o_ref):
        a = h_ref[...]
        a = jnp.maximum(
            jnp.dot(w1[...], a, preferred_element_type=jnp.float32) + c1[...], 0.0)
        a = jnp.maximum(
            jnp.dot(w2[...], a, preferred_element_type=jnp.float32) + c2[...], 0.0)
        a = jnp.maximum(
            jnp.dot(w3[...], a, preferred_element_type=jnp.float32) + c3[...], 0.0)
        a = jnp.maximum(
            jnp.dot(w4[...], a, preferred_element_type=jnp.float32) + c4[...], 0.0)
        o_ref[...] = a

    full = lambda arr: pl.BlockSpec(arr.shape, lambda i: (0, 0))
    return pl.pallas_call(
        mlp_k,
        grid=grid,
        in_specs=[
            pl.BlockSpec((2 * EMBED, blk), lambda i: (0, i)),
            full(W1t), full(b1), full(W2t), full(b2),
            full(W3t), full(b3), full(W4t), full(b4),
        ],
        out_specs=pl.BlockSpec((1, blk), lambda i: (0, i)),
        out_shape=jax.ShapeDtypeStruct((1, n_rows), jnp.float32),
    )(h_t, W1t, b1, W2t, b2, W3t, b3, W4t, b4)


def kernel(x, table, W1, b1, W2, b2, W3, b3, W4, b4):
    n_rows = x.shape[0]
    x_t = x.T
    table_flat = (table.T.reshape(2, 8, 15625, 128)
                  .transpose(0, 2, 1, 3).reshape(-1))
    h_t = _sc_gather_t(x_t, table_flat, n_rows)
    out_t = _tc_mlp_t(
        h_t,
        W1.T, b1.reshape(-1, 1),
        W2.T, b2.reshape(-1, 1),
        W3.T, b3.reshape(-1, 1),
        W4.T, b4.reshape(-1, 1),
    )
    return out_t.reshape(n_rows, 1)


# speedup vs baseline: 20.5212x; 1.0023x over previous
import functools

import jax
import jax.numpy as jnp
from jax import lax
from jax.experimental import pallas as pl
from jax.experimental.pallas import tpu as pltpu
from jax.experimental.pallas import tpu_sc as plsc

EMBED = 16
FIELD_OFFSET = 1_000_000
HALF_STRIDE = 16_000_000


def _sc_gather_t(x_t, table_flat, n_rows):
    info = plsc.get_sparse_core_info()
    nc, ns, lanes = info.num_cores, info.num_subcores, info.num_lanes
    nw = nc * ns
    mesh = plsc.VectorSubcoreMesh(core_axis_name="c", subcore_axis_name="s")

    @functools.partial(
        pl.kernel,
        mesh=mesh,
        out_type=jax.ShapeDtypeStruct((nw, n_rows), jnp.float32),
        scratch_types=[
            pltpu.VMEM((n_rows,), jnp.int32),
            pltpu.VMEM((n_rows,), jnp.int32),
            pltpu.VMEM((n_rows,), jnp.float32),
            pltpu.SemaphoreType.DMA,
        ],
        compiler_params=pltpu.CompilerParams(
            use_tc_tiling_on_sc=True, needs_layout_passes=False),
    )
    def gather_k(x_hbm, tab_hbm, out_hbm, xrow_v, idx_v, dst_v, sem):
        w = lax.axis_index("s") * nc + lax.axis_index("c")
        field = w // EMBED
        f = w % EMBED
        pltpu.sync_copy(x_hbm.at[field], xrow_v)
        base = (f // 8) * HALF_STRIDE + (f % 8) * 128
        roff = field * FIELD_OFFSET

        chunk = 512
        n_chunks = n_rows // chunk
        groups_per_chunk = chunk // lanes

        def addr_block(j):
            def inner(t, _):
                for u in range(4):
                    sl = pl.ds(j * chunk + (t * 4 + u) * lanes, lanes)
                    r = xrow_v[sl] + roff
                    idx_v[sl] = (
                        lax.shift_left(lax.shift_right_logical(r, 7), 10)
                        + (r & 127) + base)
                return 0

            lax.fori_loop(0, groups_per_chunk // 4, inner, 0)

        addr_block(0)

        def fire_chunk(j, _):
            for k in range(chunk // 128):
                pltpu.async_copy(
                    tab_hbm.at[idx_v.at[pl.ds(j * chunk + k * 128, 128)]],
                    dst_v.at[pl.ds(j * chunk + k * 128, 128)],
                    sem,
                )

            @pl.when(j < n_chunks - 1)
            def _():
                addr_block(j + 1)

            return 0

        lax.fori_loop(0, n_chunks, fire_chunk, 0)

        def drain_chunk(j, _):
            for k in range(chunk // 128):
                pltpu.make_async_copy(
                    tab_hbm.at[idx_v.at[pl.ds(0, 128)]],
                    dst_v.at[pl.ds(j * chunk + k * 128, 128)],
                    sem,
                ).wait()
            return 0

        lax.fori_loop(0, n_chunks, drain_chunk, 0)
        pltpu.sync_copy(dst_v, out_hbm.at[w])

    return gather_k(x_t, table_flat)


def _tc_mlp_t(h_t, W1t, b1, W2t, b2, W3t, b3, W4t, b4):
    n_rows = h_t.shape[1]
    blk = 8192
    grid = (n_rows // blk,)

    def mlp_k(h_ref, w1, c1, w2, c2, w3, c3, w4, c4, o_ref):
        a = h_ref[...]
        a = jnp.maximum(
            jnp.dot(w1[...], a, preferred_element_type=jnp.float32) + c1[...], 0.0)
        a = jnp.maximum(
            jnp.dot(w2[...], a, preferred_element_type=jnp.float32) + c2[...], 0.0)
        a = jnp.maximum(
            jnp.dot(w3[...], a, preferred_element_type=jnp.float32) + c3[...], 0.0)
        a = jnp.maximum(
            jnp.dot(w4[...], a, preferred_element_type=jnp.float32) + c4[...], 0.0)
        o_ref[...] = a

    full = lambda arr: pl.BlockSpec(arr.shape, lambda i: (0, 0))
    return pl.pallas_call(
        mlp_k,
        grid=grid,
        in_specs=[
            pl.BlockSpec((2 * EMBED, blk), lambda i: (0, i)),
            full(W1t), full(b1), full(W2t), full(b2),
            full(W3t), full(b3), full(W4t), full(b4),
        ],
        out_specs=pl.BlockSpec((1, blk), lambda i: (0, i)),
        out_shape=jax.ShapeDtypeStruct((1, n_rows), jnp.float32),
    )(h_t, W1t, b1, W2t, b2, W3t, b3, W4t, b4)


def kernel(x, table, W1, b1, W2, b2, W3, b3, W4, b4):
    n_rows = x.shape[0]
    x_t = x.T
    table_flat = (table.T.reshape(2, 8, 15625, 128)
                  .transpose(0, 2, 1, 3).reshape(-1))
    h_t = _sc_gather_t(x_t, table_flat, n_rows)
    out_t = _tc_mlp_t(
        h_t,
        W1.T, b1.reshape(-1, 1),
        W2.T, b2.reshape(-1, 1),
        W3.T, b3.reshape(-1, 1),
        W4.T, b4.reshape(-1, 1),
    )
    return out_t.reshape(n_rows, 1)


# speedup vs baseline: 20.6641x; 1.0070x over previous
import functools

import jax
import jax.numpy as jnp
from jax import lax
from jax.experimental import pallas as pl
from jax.experimental.pallas import tpu as pltpu
from jax.experimental.pallas import tpu_sc as plsc

EMBED = 16
FIELD_OFFSET = 1_000_000
HALF_STRIDE = 16_000_000


def _sc_gather_t(x_t, table_flat, n_rows):
    info = plsc.get_sparse_core_info()
    nc, ns, lanes = info.num_cores, info.num_subcores, info.num_lanes
    nw = nc * ns
    mesh = plsc.VectorSubcoreMesh(core_axis_name="c", subcore_axis_name="s")

    @functools.partial(
        pl.kernel,
        mesh=mesh,
        out_type=jax.ShapeDtypeStruct((nw, n_rows), jnp.float32),
        scratch_types=[
            pltpu.VMEM((n_rows,), jnp.int32),
            pltpu.VMEM((n_rows,), jnp.int32),
            pltpu.VMEM((n_rows,), jnp.float32),
            pltpu.SemaphoreType.DMA,
        ],
        compiler_params=pltpu.CompilerParams(
            use_tc_tiling_on_sc=True, needs_layout_passes=False),
    )
    def gather_k(x_hbm, tab_hbm, out_hbm, xrow_v, idx_v, dst_v, sem):
        w = lax.axis_index("s") * nc + lax.axis_index("c")
        field = w // EMBED
        f = w % EMBED
        pltpu.sync_copy(x_hbm.at[field], xrow_v)
        base = (f // 8) * HALF_STRIDE + (f % 8) * 128
        roff = field * FIELD_OFFSET

        chunk = 512
        n_chunks = n_rows // chunk
        groups_per_chunk = chunk // lanes

        def addr_block(j):
            def inner(t, _):
                for u in range(4):
                    sl = pl.ds(j * chunk + (t * 4 + u) * lanes, lanes)
                    r = xrow_v[sl] + roff
                    idx_v[sl] = (
                        lax.shift_left(lax.shift_right_logical(r, 7), 10)
                        + (r & 127) + base)
                return 0

            lax.fori_loop(0, groups_per_chunk // 4, inner, 0)

        addr_block(0)

        def fire_chunk(j, _):
            for k in range(chunk // 128):
                pltpu.async_copy(
                    tab_hbm.at[idx_v.at[pl.ds(j * chunk + k * 128, 128)]],
                    dst_v.at[pl.ds(j * chunk + k * 128, 128)],
                    sem,
                )

            @pl.when(j < n_chunks - 1)
            def _():
                addr_block(j + 1)

            return 0

        lax.fori_loop(0, n_chunks, fire_chunk, 0)
        pltpu.make_async_copy(
            tab_hbm.at[pl.ds(0, n_rows)], dst_v, sem).wait()
        pltpu.sync_copy(dst_v, out_hbm.at[w])

    return gather_k(x_t, table_flat)


def _tc_mlp_t(h_t, W1t, b1, W2t, b2, W3t, b3, W4t, b4):
    n_rows = h_t.shape[1]
    blk = 16384
    grid = (n_rows // blk,)

    def mlp_k(h_ref, w1, c1, w2, c2, w3, c3, w4, c4, o_ref):
        a = h_ref[...]
        a = jnp.maximum(
            jnp.dot(w1[...], a, preferred_element_type=jnp.float32) + c1[...], 0.0)
        a = jnp.maximum(
            jnp.dot(w2[...], a, preferred_element_type=jnp.float32) + c2[...], 0.0)
        a = jnp.maximum(
            jnp.dot(w3[...], a, preferred_element_type=jnp.float32) + c3[...], 0.0)
        a = jnp.maximum(
            jnp.dot(w4[...], a, preferred_element_type=jnp.float32) + c4[...], 0.0)
        o_ref[...] = a

    full = lambda arr: pl.BlockSpec(arr.shape, lambda i: (0, 0))
    return pl.pallas_call(
        mlp_k,
        grid=grid,
        in_specs=[
            pl.BlockSpec((2 * EMBED, blk), lambda i: (0, i)),
            full(W1t), full(b1), full(W2t), full(b2),
            full(W3t), full(b3), full(W4t), full(b4),
        ],
        out_specs=pl.BlockSpec((1, blk), lambda i: (0, i)),
        out_shape=jax.ShapeDtypeStruct((1, n_rows), jnp.float32),
    )(h_t, W1t, b1, W2t, b2, W3t, b3, W4t, b4)


def kernel(x, table, W1, b1, W2, b2, W3, b3, W4, b4):
    n_rows = x.shape[0]
    x_t = x.T
    table_flat = (table.T.reshape(2, 8, 15625, 128)
                  .transpose(0, 2, 1, 3).reshape(-1))
    h_t = _sc_gather_t(x_t, table_flat, n_rows)
    out_t = _tc_mlp_t(
        h_t,
        W1.T, b1.reshape(-1, 1),
        W2.T, b2.reshape(-1, 1),
        W3.T, b3.reshape(-1, 1),
        W4.T, b4.reshape(-1, 1),
    )
    return out_t.reshape(n_rows, 1)
